# Initial kernel scaffold; baseline (speedup 1.0000x reference)
import jax
import jax.numpy as jnp
from jax.experimental import pallas as pl


def kernel(input, embed):
    raise NotImplementedError("write your pallas kernel here")


# speedup vs baseline: 12.6716x; 12.6716x over previous
import functools

import jax
import jax.numpy as jnp
from jax import lax
from jax.experimental import pallas as pl
from jax.experimental.pallas import tpu as pltpu
from jax.experimental.pallas import tpu_sc as plsc

EMB = 256
K = 8192
NQ = 8192
BQ = 1024
NQB = NQ // BQ
NW = 32
BPW = NQ // NW


def _argmin_loss_kernel(q_ref, e_ref, e2_ref, qn_ref, ids_ref, loss_ref):
    q = q_ref[...]
    s = jax.lax.dot_general(
        e_ref[...], q + q, (((1,), (1,)), ((), ())),
        preferred_element_type=jnp.float32,
    )
    u = (s - e2_ref[...]) - qn_ref[0]
    cm = jnp.max(u, axis=0, keepdims=True)
    fiota = jax.lax.broadcasted_iota(jnp.int32, (K, BQ), 0).astype(jnp.float32)
    cand = jnp.where(u == cm, fiota, jnp.float32(K))
    bi = jnp.min(cand, axis=0, keepdims=True).astype(jnp.int32)
    ids_ref[0] = bi
    i = pl.program_id(0)
    blk = -jnp.sum(cm)
    prev = jnp.where(i == 0, jnp.zeros((1, 1), jnp.float32), loss_ref[...])
    tot = prev + blk
    loss_ref[...] = jnp.where(i == NQB - 1, tot * (1.0 / (NQ * EMB)), tot)


def _tc_argmin_loss(flatten, embed, e2, qn3):
    return pl.pallas_call(
        _argmin_loss_kernel,
        grid=(NQB,),
        in_specs=[
            pl.BlockSpec((BQ, EMB), lambda i: (i, 0)),
            pl.BlockSpec((K, EMB), lambda i: (0, 0)),
            pl.BlockSpec((K, 1), lambda i: (0, 0)),
            pl.BlockSpec((1, 1, BQ), lambda i: (i, 0, 0)),
        ],
        out_specs=[
            pl.BlockSpec((1, 1, BQ), lambda i: (i, 0, 0)),
            pl.BlockSpec((1, 1), lambda i: (0, 0)),
        ],
        out_shape=[
            jax.ShapeDtypeStruct((NQB, 1, BQ), jnp.int32),
            jax.ShapeDtypeStruct((1, 1), jnp.float32),
        ],
    )(flatten, embed, e2, qn3)


@functools.lru_cache(maxsize=1)
def _make_sc_gather():
    @functools.partial(
        pl.kernel,
        mesh=plsc.VectorSubcoreMesh(core_axis_name="c", subcore_axis_name="s"),
        out_type=jax.ShapeDtypeStruct((NQ, EMB), jnp.float32),
        scratch_types=[
            pltpu.VMEM((BPW,), jnp.int32),
            pltpu.VMEM((BPW, EMB), jnp.float32),
            pltpu.SemaphoreType.DMA,
        ],
    )
    def _sc_gather(embed_hbm, ids_hbm, out_hbm, idx_v, rows_v, sem):
        wid = lax.axis_index("s") * 2 + lax.axis_index("c")
        base = wid * BPW
        pltpu.sync_copy(ids_hbm.at[pl.ds(base, BPW)], idx_v)
        pltpu.async_copy(embed_hbm.at[idx_v], rows_v, sem).wait()
        pltpu.sync_copy(rows_v, out_hbm.at[pl.ds(base, BPW)])

    return _sc_gather


def kernel(input, embed):
    b, c, h, w = input.shape
    flatten = jnp.transpose(input, (0, 3, 2, 1)).reshape(-1, c)
    e2 = jnp.sum(embed ** 2, axis=1, keepdims=True)
    qn3 = jnp.sum(flatten ** 2, axis=1).reshape(NQB, 1, BQ)
    ids3, loss = _tc_argmin_loss(flatten, embed, e2, qn3)
    ids_flat = ids3.reshape(-1)
    qst_rows = _make_sc_gather()(embed, ids_flat)
    ids = ids_flat.reshape(b, h, w)
    quantized_st = jnp.transpose(qst_rows.reshape(b, w, h, c), (0, 3, 2, 1))
    return quantized_st, loss[0, 0], ids


# speedup vs baseline: 13.5866x; 1.0722x over previous
import functools

import jax
import jax.numpy as jnp
from jax import lax
from jax.experimental import pallas as pl
from jax.experimental.pallas import tpu as pltpu
from jax.experimental.pallas import tpu_sc as plsc

EMB = 256
K = 8192
NQ = 8192
BQ = 1024
NQB = NQ // BQ
NW = 32
BPW = NQ // NW


def _argmin_loss_kernel(x_ref, e_ref, e2_ref, qn_ref, ids_ref, loss_ref):
    x = x_ref[0]
    s = jax.lax.dot_general(
        e_ref[...], x + x, (((1,), (0,)), ((), ())),
        preferred_element_type=jnp.float32,
    )
    u = (s - e2_ref[...]) - qn_ref[0]
    cm = jnp.max(u, axis=0, keepdims=True)
    fiota = jax.lax.broadcasted_iota(jnp.int32, (K, BQ), 0).astype(jnp.float32)
    cand = jnp.where(u == cm, fiota, jnp.float32(K))
    bi = jnp.min(cand, axis=0, keepdims=True).astype(jnp.int32)
    ids_ref[0] = bi
    i = pl.program_id(0)
    blk = -jnp.sum(cm)
    prev = jnp.where(i == 0, jnp.zeros((1, 1), jnp.float32), loss_ref[...])
    tot = prev + blk
    loss_ref[...] = jnp.where(i == NQB - 1, tot * (1.0 / (NQ * EMB)), tot)


def _tc_argmin_loss(x3, embed, e2, qn3):
    return pl.pallas_call(
        _argmin_loss_kernel,
        grid=(NQB,),
        in_specs=[
            pl.BlockSpec((1, EMB, BQ), lambda i: (i, 0, 0)),
            pl.BlockSpec((K, EMB), lambda i: (0, 0)),
            pl.BlockSpec((K, 1), lambda i: (0, 0)),
            pl.BlockSpec((1, 1, BQ), lambda i: (i, 0, 0)),
        ],
        out_specs=[
            pl.BlockSpec((1, 1, BQ), lambda i: (i, 0, 0)),
            pl.BlockSpec((1, 1), lambda i: (0, 0)),
        ],
        out_shape=[
            jax.ShapeDtypeStruct((NQB, 1, BQ), jnp.int32),
            jax.ShapeDtypeStruct((1, 1), jnp.float32),
        ],
    )(x3, embed, e2, qn3)


@functools.lru_cache(maxsize=1)
def _make_sc_gather():
    @functools.partial(
        pl.kernel,
        mesh=plsc.VectorSubcoreMesh(core_axis_name="c", subcore_axis_name="s"),
        out_type=jax.ShapeDtypeStruct((NQ, EMB), jnp.float32),
        scratch_types=[
            pltpu.VMEM((BPW,), jnp.int32),
            pltpu.VMEM((BPW, EMB), jnp.float32),
            pltpu.SemaphoreType.DMA,
        ],
    )
    def _sc_gather(embed_hbm, ids_hbm, out_hbm, idx_v, rows_v, sem):
        wid = lax.axis_index("s") * 2 + lax.axis_index("c")
        base = wid * BPW
        pltpu.sync_copy(ids_hbm.at[pl.ds(base, BPW)], idx_v)
        pltpu.async_copy(embed_hbm.at[idx_v], rows_v, sem).wait()
        pltpu.sync_copy(rows_v, out_hbm.at[pl.ds(base, BPW)])

    return _sc_gather


def kernel(input, embed):
    b, c, h, w = input.shape
    x3 = input.reshape(b, c, h * w)
    e2 = jnp.sum(embed ** 2, axis=1, keepdims=True)
    qn = jnp.sum((jnp.transpose(input, (0, 3, 2, 1)).reshape(-1, c)) ** 2, axis=1)
    qn3 = jnp.transpose(qn.reshape(b, w, h), (0, 2, 1)).reshape(NQB, 1, BQ)
    ids3, loss = _tc_argmin_loss(x3, embed, e2, qn3)
    ids_flat = ids3.reshape(-1)
    qst_rows = _make_sc_gather()(embed, ids_flat)
    ids = jnp.transpose(ids3.reshape(b, h, w), (0, 2, 1))
    quantized_st = jnp.transpose(qst_rows.reshape(b, h, w, c), (0, 3, 1, 2))
    return quantized_st, loss[0, 0], ids


# speedup vs baseline: 15.6348x; 1.1507x over previous
import functools

import jax
import jax.numpy as jnp
from jax import lax
from jax.experimental import pallas as pl
from jax.experimental.pallas import tpu as pltpu
from jax.experimental.pallas import tpu_sc as plsc

EMB = 256
K = 8192
NQ = 8192
BQ = 1024
NQB = NQ // BQ
NW = 32
BPW = NQ // NW


def _argmin_loss_kernel(x_ref, e_ref, e2_ref, qn_ref, ids_ref, loss_ref):
    x = x_ref[0]
    s = jax.lax.dot_general(
        e_ref[...], x + x, (((1,), (0,)), ((), ())),
        preferred_element_type=jnp.float32,
    )
    u = (s - e2_ref[...]) - qn_ref[0]
    cm = jnp.max(u, axis=0, keepdims=True)
    bi = jnp.argmax(u, axis=0).astype(jnp.int32).reshape(1, BQ)
    ids_ref[0] = bi
    i = pl.program_id(0)
    blk = -jnp.sum(cm)
    prev = jnp.where(i == 0, jnp.zeros((1, 1), jnp.float32), loss_ref[...])
    tot = prev + blk
    loss_ref[...] = jnp.where(i == NQB - 1, tot * (1.0 / (NQ * EMB)), tot)


def _tc_argmin_loss(x3, embed, e2, qn3):
    return pl.pallas_call(
        _argmin_loss_kernel,
        grid=(NQB,),
        in_specs=[
            pl.BlockSpec((1, EMB, BQ), lambda i: (i, 0, 0)),
            pl.BlockSpec((K, EMB), lambda i: (0, 0)),
            pl.BlockSpec((K, 1), lambda i: (0, 0)),
            pl.BlockSpec((1, 1, BQ), lambda i: (i, 0, 0)),
        ],
        out_specs=[
            pl.BlockSpec((1, 1, BQ), lambda i: (i, 0, 0)),
            pl.BlockSpec((1, 1), lambda i: (0, 0)),
        ],
        out_shape=[
            jax.ShapeDtypeStruct((NQB, 1, BQ), jnp.int32),
            jax.ShapeDtypeStruct((1, 1), jnp.float32),
        ],
    )(x3, embed, e2, qn3)


@functools.lru_cache(maxsize=1)
def _make_sc_gather():
    @functools.partial(
        pl.kernel,
        mesh=plsc.VectorSubcoreMesh(core_axis_name="c", subcore_axis_name="s"),
        out_type=jax.ShapeDtypeStruct((NQ, EMB), jnp.float32),
        scratch_types=[
            pltpu.VMEM((BPW,), jnp.int32),
            pltpu.VMEM((BPW, EMB), jnp.float32),
            pltpu.SemaphoreType.DMA,
        ],
    )
    def _sc_gather(embed_hbm, ids_hbm, out_hbm, idx_v, rows_v, sem):
        wid = lax.axis_index("s") * 2 + lax.axis_index("c")
        base = wid * BPW
        pltpu.sync_copy(ids_hbm.at[pl.ds(base, BPW)], idx_v)
        pltpu.async_copy(embed_hbm.at[idx_v], rows_v, sem).wait()
        pltpu.sync_copy(rows_v, out_hbm.at[pl.ds(base, BPW)])

    return _sc_gather


def kernel(input, embed):
    b, c, h, w = input.shape
    x3 = input.reshape(b, c, h * w)
    e2 = jnp.sum(embed ** 2, axis=1, keepdims=True)
    qn = jnp.sum((jnp.transpose(input, (0, 3, 2, 1)).reshape(-1, c)) ** 2, axis=1)
    qn3 = jnp.transpose(qn.reshape(b, w, h), (0, 2, 1)).reshape(NQB, 1, BQ)
    ids3, loss = _tc_argmin_loss(x3, embed, e2, qn3)
    ids_flat = ids3.reshape(-1)
    qst_rows = _make_sc_gather()(embed, ids_flat)
    ids = jnp.transpose(ids3.reshape(b, h, w), (0, 2, 1))
    quantized_st = jnp.transpose(qst_rows.reshape(b, h, w, c), (0, 3, 1, 2))
    return quantized_st, loss[0, 0], ids
